# Initial kernel scaffold; baseline (speedup 1.0000x reference)
import jax
import jax.numpy as jnp
from jax.experimental import pallas as pl


def kernel(x, edge_index, W_sheaf, W1, b1, W2, b2, W3, b3, W4, b4):
    raise NotImplementedError("write your pallas kernel here")


# speedup vs baseline: 5.0081x; 5.0081x over previous
import functools

import jax
import jax.numpy as jnp
from jax import lax
from jax.experimental import pallas as pl
from jax.experimental.pallas import tpu as pltpu
from jax.experimental.pallas import tpu_sc as plsc

N = 10000
D = 128
E = 320000
CUM = 256
HID = 128
NC = 2
NS = 16
NW = NC * NS
EPW = E // NW
K = 80
NCH = EPW // K
NP = 10240
RPT = NP // NS
RZB = 128
BLK = 1000

_mesh = plsc.VectorSubcoreMesh(core_axis_name="c", subcore_axis_name="s")
_F32 = jnp.float32


def _make_seg():
    scratch = [
        pltpu.VMEM((NCH, K), jnp.int32),
        pltpu.VMEM((NCH, K), jnp.int32),
        pltpu.VMEM((K,), jnp.int32),
        pltpu.VMEM((K,), jnp.int32),
        pltpu.VMEM((K, D), _F32),
        pltpu.VMEM_SHARED((NP, D), _F32),
        pltpu.SemaphoreType.DMA,
    ]

    def body(v_hbm, src_hbm, dst_hbm, z_hbm, part_hbm,
             src_v, dst_v, src_ck, dst_ck, rows_v, acc, sem):
        c = lax.axis_index("c")
        s = lax.axis_index("s")
        wid = c * NS + s
        pltpu.sync_copy(src_hbm.at[wid], src_v)
        pltpu.sync_copy(dst_hbm.at[wid], dst_v)
        pltpu.sync_copy(z_hbm, acc.at[pl.ds(s * RPT, RPT)])
        plsc.subcore_barrier()

        def chunk(ch, carry):
            for g in range(K // 16):
                sl = pl.ds(g * 16, 16)
                src_ck[sl] = src_v[ch, sl]
                dst_ck[sl] = dst_v[ch, sl]
            pltpu.async_copy(v_hbm.at[src_ck], rows_v, sem).wait()
            pltpu.sync_copy(rows_v, acc.at[dst_ck], add=True)
            return carry

        lax.fori_loop(0, NCH, chunk, 0)
        plsc.subcore_barrier()

        rows = pl.ds(s * RPT, RPT)
        pltpu.sync_copy(acc.at[rows], part_hbm.at[c, rows])

    return pl.kernel(body, out_type=jax.ShapeDtypeStruct((NC, NP, D), _F32),
                     mesh=_mesh, scratch_types=scratch)


def _make_deg():
    scratch = [
        pltpu.VMEM((NCH, K), jnp.int32),
        pltpu.VMEM((K,), jnp.int32),
        pltpu.VMEM((K, D), _F32),
        pltpu.VMEM_SHARED((NP, D), _F32),
    ]

    def body(dst_hbm, z_hbm, degp_hbm, dst_v, dst_ck, ones_v, accd):
        c = lax.axis_index("c")
        s = lax.axis_index("s")
        wid = c * NS + s
        pltpu.sync_copy(dst_hbm.at[wid], dst_v)
        pltpu.sync_copy(z_hbm, accd.at[pl.ds(s * RPT, RPT)])
        o16 = jnp.ones((16,), _F32)

        def orow(r, carry):
            for f in range(D // 16):
                ones_v[r, pl.ds(f * 16, 16)] = o16
            return carry

        lax.fori_loop(0, K, orow, 0)
        plsc.subcore_barrier()

        def chunk(ch, carry):
            for g in range(K // 16):
                sl = pl.ds(g * 16, 16)
                dst_ck[sl] = dst_v[ch, sl]
            pltpu.sync_copy(ones_v, accd.at[dst_ck], add=True)
            return carry

        lax.fori_loop(0, NCH, chunk, 0)
        plsc.subcore_barrier()

        rows = pl.ds(s * RPT, RPT)
        pltpu.sync_copy(accd.at[rows], degp_hbm.at[c, rows])

    return pl.kernel(body, out_type=jax.ShapeDtypeStruct((NC, NP, D), _F32),
                     mesh=_mesh, scratch_types=scratch)


_seg = _make_seg()
_deg = _make_deg()


@functools.partial(
    pl.kernel,
    out_type=jax.ShapeDtypeStruct((E,), _F32),
    mesh=_mesh,
    scratch_types=[
        pltpu.VMEM((NCH, K), jnp.int32),
        pltpu.VMEM((NCH, K), jnp.int32),
        pltpu.VMEM((K,), jnp.int32),
        pltpu.VMEM((K,), jnp.int32),
        pltpu.VMEM((K, D), _F32),
        pltpu.VMEM((K, D), _F32),
        pltpu.VMEM((K,), _F32),
        pltpu.SemaphoreType.DMA,
        pltpu.SemaphoreType.DMA,
    ],
)
def _r_kernel(x_hbm, y_hbm, src_hbm, dst_hbm, r_hbm,
              src_v, dst_v, src_ck, dst_ck, ya, xb, rbuf, sem_a, sem_b):
    c = lax.axis_index("c")
    s = lax.axis_index("s")
    wid = c * NS + s
    pltpu.sync_copy(src_hbm.at[wid], src_v)
    pltpu.sync_copy(dst_hbm.at[wid], dst_v)

    def chunk(ch, carry):
        for g in range(K // 16):
            sl = pl.ds(g * 16, 16)
            src_ck[sl] = src_v[ch, sl]
            dst_ck[sl] = dst_v[ch, sl]
        cp_a = pltpu.async_copy(y_hbm.at[src_ck], ya, sem_a)
        cp_b = pltpu.async_copy(x_hbm.at[dst_ck], xb, sem_b)
        cp_a.wait()
        cp_b.wait()

        lane = lax.broadcasted_iota(jnp.int32, (16,), 0)

        def group(g, carry2):
            def edge(j, r16):
                e = g * 16 + j
                acc = ya[e, pl.ds(0, 16)] * xb[e, pl.ds(0, 16)]
                for f in range(1, D // 16):
                    acc = acc + (ya[e, pl.ds(f * 16, 16)]
                                 * xb[e, pl.ds(f * 16, 16)])
                for sh in (8, 4, 2, 1):
                    acc = acc + acc.at[lane ^ sh].get(
                        mode="promise_in_bounds")
                return jnp.where(lane == j, acc, r16)

            r16 = lax.fori_loop(0, 16, edge, jnp.zeros((16,), _F32))
            rbuf[pl.ds(g * 16, 16)] = 1.0 / (1.0 + jnp.exp(-r16))
            return carry2

        lax.fori_loop(0, K // 16, group, 0)
        pltpu.sync_copy(rbuf, r_hbm.at[pl.ds(wid * EPW + ch * K, K)])
        return carry

    lax.fori_loop(0, NCH, chunk, 0)


def _dot(a, b):
    return jnp.dot(a, b, preferred_element_type=_F32)


def _dinv_of(degp_ref):
    d = degp_ref[0, :, 0:1] + degp_ref[1, :, 0:1]
    return 1.0 / jnp.maximum(d, 1.0)


def _mm_body(x_ref, w_ref, o_ref):
    o_ref[...] = _dot(x_ref[...], w_ref[...])


def _comb_body(part_ref, degp_ref, o_ref):
    o_ref[...] = (part_ref[0] + part_ref[1]) * _dinv_of(degp_ref)


def _gcn1_body(h2_ref, p3_ref, degp_ref, w1_ref, b1_ref, oa_ref, ob_ref):
    h3 = (p3_ref[0] + p3_ref[1]) * _dinv_of(degp_ref)
    o = _dot(h2_ref[...], w1_ref[0:D]) + _dot(h3, w1_ref[D:CUM]) + b1_ref[...]
    o = jnp.maximum(o, 0.0)
    oa_ref[...] = o[:, 0:D]
    ob_ref[...] = o[:, D:CUM]


def _tail_body(qa_ref, qb_ref, degp_ref, w2_ref, b2_ref, w3_ref, b3_ref,
               w4_ref, b4_ref, o_ref):
    dinv = _dinv_of(degp_ref)
    agg_a = (qa_ref[0] + qa_ref[1]) * dinv
    agg_b = (qb_ref[0] + qb_ref[1]) * dinv
    o2 = _dot(agg_a, w2_ref[0:D]) + _dot(agg_b, w2_ref[D:CUM]) + b2_ref[...]
    o2 = jnp.maximum(o2, 0.0)
    o3 = jnp.maximum(_dot(o2, w3_ref[...]) + b3_ref[...], 0.0)
    o_ref[...] = _dot(o3, w4_ref[...]) + b4_ref[...]


def _rows(i):
    return (i, 0)


def _rows3(i):
    return (0, i, 0)


def _fix(i):
    return (0, 0)


_GRID = N // BLK

_part_spec = pl.BlockSpec((NC, BLK, D), _rows3)
_degp_spec = pl.BlockSpec((NC, BLK, D), _rows3)
_nd_spec = pl.BlockSpec((BLK, D), _rows)


def _mm(x, w):
    return pl.pallas_call(
        _mm_body,
        grid=(_GRID,),
        in_specs=[_nd_spec, pl.BlockSpec((D, D), _fix)],
        out_specs=_nd_spec,
        out_shape=jax.ShapeDtypeStruct((N, D), _F32),
    )(x, w)


def _combine(part, degp):
    return pl.pallas_call(
        _comb_body,
        grid=(_GRID,),
        in_specs=[_part_spec, _degp_spec],
        out_specs=_nd_spec,
        out_shape=jax.ShapeDtypeStruct((N, D), _F32),
    )(part, degp)


def _gcn1(h2, p3, degp, w1, b1):
    return pl.pallas_call(
        _gcn1_body,
        grid=(_GRID,),
        in_specs=[
            _nd_spec, _part_spec, _degp_spec,
            pl.BlockSpec((CUM, CUM), _fix), pl.BlockSpec((1, CUM), _fix),
        ],
        out_specs=[_nd_spec, _nd_spec],
        out_shape=[jax.ShapeDtypeStruct((N, D), _F32)] * 2,
    )(h2, p3, degp, w1, b1)


def _tail(qa, qb, degp, w2, b2, w3, b3, w4, b4):
    return pl.pallas_call(
        _tail_body,
        grid=(_GRID,),
        in_specs=[
            _part_spec, _part_spec, _degp_spec,
            pl.BlockSpec((CUM, CUM), _fix), pl.BlockSpec((1, CUM), _fix),
            pl.BlockSpec((CUM, HID), _fix), pl.BlockSpec((1, HID), _fix),
            pl.BlockSpec((HID, HID), _fix), pl.BlockSpec((1, HID), _fix),
        ],
        out_specs=_nd_spec,
        out_shape=jax.ShapeDtypeStruct((N, D), _F32),
    )(qa, qb, degp, w2, b2, w3, b3, w4, b4)


def kernel(x, edge_index, W_sheaf, W1, b1, W2, b2, W3, b3, W4, b4):
    src = edge_index[0].reshape(NW, NCH, K)
    dst = edge_index[1].reshape(NW, NCH, K)

    z = jnp.zeros((RPT, D), _F32)

    y = _mm(x, W_sheaf)
    R = _r_kernel(x, y, src, dst)

    degp = _deg(dst, z)
    p1 = _seg(x, src, dst, z)
    h1 = _combine(p1, degp)
    p2 = _seg(h1, src, dst, z)
    h2 = _combine(p2, degp)
    p3 = _seg(h2, src, dst, z)
    out1a, out1b = _gcn1(h2, p3, degp, W1.reshape(CUM, CUM),
                         b1.reshape(1, CUM))
    qa = _seg(out1a, src, dst, z)
    qb = _seg(out1b, src, dst, z)
    out = _tail(qa, qb, degp, W2.reshape(CUM, CUM), b2.reshape(1, CUM),
                W3.reshape(CUM, HID), b3.reshape(1, HID),
                W4.reshape(HID, HID), b4.reshape(1, HID))
    return (out, R)
